# baseline (speedup 1.0000x reference)
import functools

import jax
import jax.numpy as jnp
from jax import lax
from jax.experimental import pallas as pl
from jax.experimental.pallas import tpu as pltpu
from jax.experimental.pallas import tpu_sc as plsc

_CHUNK = 128


def kernel(node_idx, table):
    B = node_idx.shape[0]
    V, D = table.shape
    info = plsc.get_sparse_core_info()
    NC, NS = info.num_cores, info.num_subcores
    NW = NC * NS
    assert B % (8 * NW) == 0
    b_per_w = B // NW
    n_chunks = b_per_w // _CHUNK
    assert n_chunks * _CHUNK == b_per_w

    idx3 = node_idx.astype(jnp.int32).reshape(NW, n_chunks, _CHUNK)

    mesh = plsc.VectorSubcoreMesh(core_axis_name="c", subcore_axis_name="s")

    @functools.partial(
        pl.kernel,
        mesh=mesh,
        out_type=jax.ShapeDtypeStruct((B, D), jnp.float32),
        scratch_types=[
            pltpu.VMEM((n_chunks, _CHUNK), jnp.int32),
            pltpu.VMEM((b_per_w, D), jnp.float32),
            pltpu.SemaphoreType.DMA,
        ],
        compiler_params=pltpu.CompilerParams(use_tc_tiling_on_sc=False),
    )
    def body(idx_hbm, table_hbm, out_hbm, idx_v, rows_v, sem):
        wid = lax.axis_index("s") * NC + lax.axis_index("c")
        base = wid * b_per_w
        pltpu.sync_copy(idx_hbm.at[wid], idx_v)
        copies = [
            pltpu.async_copy(
                table_hbm.at[idx_v.at[j]],
                rows_v.at[pl.ds(j * _CHUNK, _CHUNK)],
                sem,
            )
            for j in range(n_chunks)
        ]
        for c in copies:
            c.wait()
        pltpu.sync_copy(rows_v, out_hbm.at[pl.ds(base, b_per_w)])

    return body(idx3, table)


# speedup vs baseline: 1.6912x; 1.6912x over previous
import functools

import jax
import jax.numpy as jnp
from jax import lax
from jax.experimental import pallas as pl
from jax.experimental.pallas import tpu as pltpu
from jax.experimental.pallas import tpu_sc as plsc

_FIRE = 128


def kernel(node_idx, table):
    B = node_idx.shape[0]
    V, D = table.shape
    info = plsc.get_sparse_core_info()
    NC, NS = info.num_cores, info.num_subcores
    NW = NC * NS
    b_per_w = B // NW

    idx2 = node_idx.astype(jnp.int32).reshape(NW, b_per_w)

    mesh = plsc.VectorSubcoreMesh(core_axis_name="c", subcore_axis_name="s")

    @functools.partial(
        pl.kernel,
        mesh=mesh,
        out_type=jax.ShapeDtypeStruct((B, D), jnp.float32),
        scratch_types=[
            pltpu.VMEM((b_per_w,), jnp.int32),
            pltpu.SMEM((b_per_w,), jnp.int32),
            pltpu.VMEM((b_per_w, D), jnp.float32),
            pltpu.SemaphoreType.DMA,
        ],
    )
    def body(idx_hbm, table_hbm, out_hbm, idx_v, idx_s, rows_v, sem):
        wid = lax.axis_index("s") * NC + lax.axis_index("c")
        base = wid * b_per_w
        pltpu.sync_copy(idx_hbm.at[wid], idx_v)
        for f in range(b_per_w // _FIRE):
            copies = []
            for g in range(_FIRE // 16):
                jbase = f * _FIRE + g * 16
                vec = idx_v[pl.ds(jbase, 16)]
                for l in range(16):
                    r = vec[l]
                    copies.append(pltpu.async_copy(
                        table_hbm.at[pl.ds(r, 1)],
                        rows_v.at[pl.ds(jbase + l, 1)], sem))
            for c in copies:
                c.wait()
        pltpu.sync_copy(rows_v, out_hbm.at[pl.ds(base, b_per_w)])

    return body(idx2, table)


# speedup vs baseline: 1.7155x; 1.0144x over previous
import functools

import jax
import jax.numpy as jnp
from jax import lax
from jax.experimental import pallas as pl
from jax.experimental.pallas import tpu as pltpu
from jax.experimental.pallas import tpu_sc as plsc

_GROUPS_PER_BATCH = 8


def kernel(node_idx, table):
    B = node_idx.shape[0]
    V, D = table.shape
    info = plsc.get_sparse_core_info()
    NC, NS = info.num_cores, info.num_subcores
    NW = NC * NS
    b_per_w = B // NW
    n_groups = b_per_w // 16
    n_batches = n_groups // _GROUPS_PER_BATCH
    rows_per_batch = _GROUPS_PER_BATCH * 16

    idx2 = node_idx.astype(jnp.int32).reshape(NW, b_per_w)

    mesh = plsc.VectorSubcoreMesh(core_axis_name="c", subcore_axis_name="s")

    @functools.partial(
        pl.kernel,
        mesh=mesh,
        out_type=jax.ShapeDtypeStruct((B, D), jnp.float32),
        scratch_types=[
            pltpu.VMEM((b_per_w,), jnp.int32),
            pltpu.VMEM((b_per_w, D), jnp.float32),
            pltpu.SemaphoreType.DMA,
        ],
    )
    def body(idx_hbm, table_hbm, out_hbm, idx_v, rows_v, sem):
        wid = lax.axis_index("s") * NC + lax.axis_index("c")
        base = wid * b_per_w
        pltpu.sync_copy(idx_hbm.at[wid], idx_v)

        def fire_group(g, carry):
            vec = idx_v[pl.ds(g * 16, 16)]
            for l in range(16):
                r = vec[l]
                pltpu.async_copy(table_hbm.at[pl.ds(r, 1)],
                                 rows_v.at[pl.ds(g * 16 + l, 1)], sem)
            return carry

        for f in range(n_batches):
            lax.fori_loop(f * _GROUPS_PER_BATCH, (f + 1) * _GROUPS_PER_BATCH,
                          fire_group, 0)
            pltpu.make_async_copy(
                table_hbm.at[pl.ds(0, rows_per_batch)],
                rows_v.at[pl.ds(f * rows_per_batch, rows_per_batch)],
                sem).wait()

        pltpu.sync_copy(rows_v, out_hbm.at[pl.ds(base, b_per_w)])

    return body(idx2, table)
